# baseline (speedup 1.0000x reference)
import jax
import jax.numpy as jnp
from jax import lax
from jax.experimental import pallas as pl
from jax.experimental.pallas import tpu as pltpu
from jax.experimental.pallas import tpu_sc as plsc

_NC, _NS = 2, 16
_NW = _NC * _NS
_CHUNK = 128


def _sc_gather(emb, idx3d):
    nw, chunks_per_w, chunk = idx3d.shape
    n_chunks = nw * chunks_per_w
    E = emb.shape[1]

    def body(emb_hbm, idx_hbm, out_hbm, idx_v,
             buf0, buf1, g0, g1, o0, o1):
        wid = lax.axis_index("s") * _NC + lax.axis_index("c")
        base = wid * chunks_per_w
        pltpu.sync_copy(idx_hbm.at[wid], idx_v)
        bufs, gsems, osems = (buf0, buf1), (g0, g1), (o0, o1)

        def gather(c, j):
            return pltpu.make_async_copy(
                emb_hbm.at[idx_v.at[c]], bufs[j], gsems[j])

        def putout(c, j):
            return pltpu.make_async_copy(
                bufs[j], out_hbm.at[base + c], osems[j])

        gather(0, 0).start()
        gather(1, 1).start()

        def pair_step(p, carry):
            for j in range(2):
                c = 2 * p + j
                gather(c, j).wait()
                putout(c, j).start()
                putout(c, j).wait()

                @pl.when(c + 2 < chunks_per_w)
                def _():
                    gather(c + 2, j).start()
            return carry

        lax.fori_loop(0, chunks_per_w // 2, pair_step, 0)

    f = pl.kernel(
        body,
        out_type=jax.ShapeDtypeStruct((n_chunks, chunk, E), jnp.float32),
        mesh=plsc.VectorSubcoreMesh(core_axis_name="c", subcore_axis_name="s"),
        scratch_types=[
            pltpu.VMEM((chunks_per_w, chunk), jnp.int32),
            pltpu.VMEM((chunk, E), jnp.float32),
            pltpu.VMEM((chunk, E), jnp.float32),
            pltpu.SemaphoreType.DMA,
            pltpu.SemaphoreType.DMA,
            pltpu.SemaphoreType.DMA,
            pltpu.SemaphoreType.DMA,
        ],
    )
    return f(emb, idx3d)


def _sig(x):
    return 0.5 * jnp.tanh(0.5 * x) + 0.5


def _gates(gg, c, H_):
    i, f, g, o = jnp.split(gg, 4, axis=1)
    ti = jnp.tanh(i)
    tf = jnp.tanh(f)
    tg = jnp.tanh(g)
    to = jnp.tanh(o)
    c_new = 0.5 * ((1.0 + tf) * c + (1.0 + ti) * tg)
    h2 = (1.0 + to) * jnp.tanh(c_new)
    return c_new, h2


def _lstm_seg(embeds, w0, w1, b0, b1, lwt, lb, state):
    L_, B_, E_ = embeds.shape
    H_ = w0.shape[1] // 4
    C_ = lwt.shape[1]

    def body(e_ref, w0_ref, w1_ref, b0_ref, b1_ref, lw_ref, lb_ref,
             h0_in, c0_in, h1_in, c1_in,
             h0_out, c0_out, h1_out, c1_out, sig_ref,
             h0, c0, h1, c1):
        t = pl.program_id(0)

        @pl.when(t == 0)
        def _():
            h0[...] = h0_in[...]
            c0[...] = c0_in[...]
            h1[...] = h1_in[...]
            c1[...] = c1_in[...]

        h0p = h0[...]

        @pl.when(t > 0)
        def _():
            cat1 = jnp.concatenate([h0p, h1[...]], axis=1)
            g1 = jnp.dot(cat1, w1_ref[...],
                         preferred_element_type=jnp.float32) + b1_ref[...]
            c1n, h1n2 = _gates(g1, c1[...], H_)
            h1[...] = h1n2.astype(jnp.bfloat16)
            c1[...] = c1n

        @pl.when(t < L_)
        def _():
            e = e_ref[0].astype(jnp.bfloat16)
            cat0 = jnp.concatenate([e, h0p], axis=1)
            g0 = jnp.dot(cat0, w0_ref[...],
                         preferred_element_type=jnp.float32) + b0_ref[...]
            c0n, h0n2 = _gates(g0, c0[...], H_)
            h0[...] = h0n2.astype(jnp.bfloat16)
            c0[...] = c0n

        @pl.when(t == L_)
        def _():
            h0_out[...] = h0[...]
            c0_out[...] = c0[...]
            h1_out[...] = h1[...]
            c1_out[...] = c1[...]
            logits = jnp.dot(h1[...], lw_ref[...].astype(jnp.bfloat16),
                             preferred_element_type=jnp.float32) + lb_ref[...]
            sig_ref[...] = _sig(logits)

    full = lambda shape: pl.BlockSpec(shape, lambda t: (0,) * len(shape))
    return pl.pallas_call(
        body,
        grid=(L_ + 1,),
        in_specs=[
            pl.BlockSpec((1, B_, E_), lambda t: (jnp.minimum(t, L_ - 1), 0, 0)),
            full(w0.shape), full(w1.shape), full(b0.shape), full(b1.shape),
            full(lwt.shape), full(lb.shape),
            full((B_, H_)), full((B_, H_)), full((B_, H_)), full((B_, H_)),
        ],
        out_specs=[full((B_, H_))] * 4 + [full((B_, C_))],
        out_shape=[jax.ShapeDtypeStruct((B_, H_), jnp.bfloat16),
                   jax.ShapeDtypeStruct((B_, H_), jnp.float32),
                   jax.ShapeDtypeStruct((B_, H_), jnp.bfloat16),
                   jax.ShapeDtypeStruct((B_, H_), jnp.float32),
                   jax.ShapeDtypeStruct((B_, C_), jnp.float32)],
        scratch_shapes=[pltpu.VMEM((B_, H_), jnp.bfloat16),
                        pltpu.VMEM((B_, H_), jnp.float32),
                        pltpu.VMEM((B_, H_), jnp.bfloat16),
                        pltpu.VMEM((B_, H_), jnp.float32)],
    )(embeds, w0, w1, b0, b1, lwt, lb, *state)


_NSEG = 5


def kernel(x, emb, W_ih0, W_hh0, b_ih0, b_hh0, W_ih1, W_hh1, b_ih1, b_hh1,
           lin_w, lin_b):
    B_, L_ = x.shape
    E_ = emb.shape[1]
    H_ = W_hh0.shape[1]
    Ls = L_ // _NSEG

    xt = x.T
    cs = jnp.concatenate([jnp.full((H_,), 0.5), jnp.full((H_,), 0.5),
                          jnp.ones((H_,)), jnp.full((H_,), 0.5)])
    w0 = (jnp.concatenate([W_ih0.T, 0.5 * W_hh0.T], axis=0)
          * cs).astype(jnp.bfloat16)
    w1 = (0.5 * jnp.concatenate([W_ih1.T, W_hh1.T], axis=0)
          * cs).astype(jnp.bfloat16)
    b0 = ((b_ih0 + b_hh0) * cs).reshape(1, -1)
    b1 = ((b_ih1 + b_hh1) * cs).reshape(1, -1)
    lwt = 0.5 * lin_w.T
    lb = lin_b.reshape(1, -1)

    segs = []
    for s in range(_NSEG):
        idx3d = xt[s * Ls:(s + 1) * Ls].reshape(_NW, -1, _CHUNK)
        segs.append(_sc_gather(emb, idx3d).reshape(Ls, B_, E_))

    zh = jnp.zeros((B_, H_), jnp.bfloat16)
    zc = jnp.zeros((B_, H_), jnp.float32)
    state = (zh, zc, zh, zc)
    for s in range(_NSEG):
        *state, sig = _lstm_seg(segs[s], w0, w1, b0, b1, lwt, lb, state)
    return sig[:, -1]


# speedup vs baseline: 1.1400x; 1.1400x over previous
import jax
import jax.numpy as jnp
from jax import lax
from jax.experimental import pallas as pl
from jax.experimental.pallas import tpu as pltpu
from jax.experimental.pallas import tpu_sc as plsc

_NC, _NS = 2, 16
_NW = _NC * _NS
_CHUNK = 128


def _sc_gather(emb, idx3d):
    nw, chunks_per_w, chunk = idx3d.shape
    n_chunks = nw * chunks_per_w
    E = emb.shape[1]

    def body(emb_hbm, idx_hbm, out_hbm, idx_v,
             buf0, buf1, g0, g1, o0, o1):
        wid = lax.axis_index("s") * _NC + lax.axis_index("c")
        base = wid * chunks_per_w
        pltpu.sync_copy(idx_hbm.at[wid], idx_v)
        bufs, gsems, osems = (buf0, buf1), (g0, g1), (o0, o1)

        def gather(c, j):
            return pltpu.make_async_copy(
                emb_hbm.at[idx_v.at[c]], bufs[j], gsems[j])

        def putout(c, j):
            return pltpu.make_async_copy(
                bufs[j], out_hbm.at[base + c], osems[j])

        gather(0, 0).start()
        gather(1, 1).start()

        def pair_step(p, carry):
            for j in range(2):
                c = 2 * p + j
                gather(c, j).wait()
                putout(c, j).start()
                putout(c, j).wait()

                @pl.when(c + 2 < chunks_per_w)
                def _():
                    gather(c + 2, j).start()
            return carry

        lax.fori_loop(0, chunks_per_w // 2, pair_step, 0)

    f = pl.kernel(
        body,
        out_type=jax.ShapeDtypeStruct((n_chunks, chunk, E), jnp.float32),
        mesh=plsc.VectorSubcoreMesh(core_axis_name="c", subcore_axis_name="s"),
        scratch_types=[
            pltpu.VMEM((chunks_per_w, chunk), jnp.int32),
            pltpu.VMEM((chunk, E), jnp.float32),
            pltpu.VMEM((chunk, E), jnp.float32),
            pltpu.SemaphoreType.DMA,
            pltpu.SemaphoreType.DMA,
            pltpu.SemaphoreType.DMA,
            pltpu.SemaphoreType.DMA,
        ],
    )
    return f(emb, idx3d)


def _sig(x):
    return 0.5 * jnp.tanh(0.5 * x) + 0.5


def _gates(gg, c, H_):
    i, f, g, o = jnp.split(gg, 4, axis=1)
    ti = jnp.tanh(i)
    tf = jnp.tanh(f)
    tg = jnp.tanh(g)
    to = jnp.tanh(o)
    c_new = 0.5 * ((1.0 + tf) * c + (1.0 + ti) * tg)
    h2 = (1.0 + to) * jnp.tanh(c_new)
    return c_new, h2


def _lstm_seg(embeds, w0, w1, b0, b1, lwt, lb, state, last):
    L_, B_, E_ = embeds.shape
    H_ = w0.shape[1] // 4
    C_ = lwt.shape[1]
    n_iter = L_ + 1 if last else L_

    def body(e_ref, w0_ref, w1_ref, b0_ref, b1_ref, lw_ref, lb_ref,
             h0_in, c0_in, h1_in, c1_in,
             h0_out, c0_out, h1_out, c1_out, sig_ref,
             h0, c0, h1, c1):
        t = pl.program_id(0)

        @pl.when(t == 0)
        def _():
            h0[...] = h0_in[...]
            c0[...] = c0_in[...]
            h1[...] = h1_in[...]
            c1[...] = c1_in[...]

        h0p = h0[...]

        cat1 = jnp.concatenate([h0p, h1[...]], axis=1)
        g1 = jnp.dot(cat1, w1_ref[...],
                     preferred_element_type=jnp.float32) + b1_ref[...]
        c1n, h1n2 = _gates(g1, c1[...], H_)
        h1n_b = h1n2.astype(jnp.bfloat16)
        h1[...] = h1n_b
        c1[...] = c1n

        def layer0():
            e = e_ref[0].astype(jnp.bfloat16)
            cat0 = jnp.concatenate([e, h0p], axis=1)
            g0 = jnp.dot(cat0, w0_ref[...],
                         preferred_element_type=jnp.float32) + b0_ref[...]
            c0n, h0n2 = _gates(g0, c0[...], H_)
            h0n_b = h0n2.astype(jnp.bfloat16)
            h0[...] = h0n_b
            c0[...] = c0n
            return c0n, h0n_b

        if last:
            @pl.when(t < L_)
            def _():
                layer0()

            @pl.when(t == L_)
            def _():
                h0_out[...] = h0[...]
                c0_out[...] = c0[...]
                h1_out[...] = h1n_b
                c1_out[...] = c1n
                logits = jnp.dot(h1n2, lw_ref[...],
                                 preferred_element_type=jnp.float32) + lb_ref[...]
                sig_ref[...] = _sig(logits)
        else:
            c0n, h0n_b = layer0()

            @pl.when(t == L_ - 1)
            def _():
                h0_out[...] = h0n_b
                c0_out[...] = c0n
                h1_out[...] = h1n_b
                c1_out[...] = c1n
                sig_ref[...] = jnp.zeros_like(sig_ref)

    full = lambda shape: pl.BlockSpec(shape, lambda t: (0,) * len(shape))
    e_index = ((lambda t: (jnp.minimum(t, L_ - 1), 0, 0)) if last
               else (lambda t: (t, 0, 0)))
    return pl.pallas_call(
        body,
        grid=(n_iter,),
        in_specs=[
            pl.BlockSpec((1, B_, E_), e_index),
            full(w0.shape), full(w1.shape), full(b0.shape), full(b1.shape),
            full(lwt.shape), full(lb.shape),
            full((B_, H_)), full((B_, H_)), full((B_, H_)), full((B_, H_)),
        ],
        out_specs=[full((B_, H_))] * 4 + [full((B_, C_))],
        out_shape=[jax.ShapeDtypeStruct((B_, H_), jnp.bfloat16),
                   jax.ShapeDtypeStruct((B_, H_), jnp.float32),
                   jax.ShapeDtypeStruct((B_, H_), jnp.bfloat16),
                   jax.ShapeDtypeStruct((B_, H_), jnp.float32),
                   jax.ShapeDtypeStruct((B_, C_), jnp.float32)],
        scratch_shapes=[pltpu.VMEM((B_, H_), jnp.bfloat16),
                        pltpu.VMEM((B_, H_), jnp.float32),
                        pltpu.VMEM((B_, H_), jnp.bfloat16),
                        pltpu.VMEM((B_, H_), jnp.float32)],
    )(embeds, w0, w1, b0, b1, lwt, lb, *state)


_NSEG = 5


def kernel(x, emb, W_ih0, W_hh0, b_ih0, b_hh0, W_ih1, W_hh1, b_ih1, b_hh1,
           lin_w, lin_b):
    B_, L_ = x.shape
    E_ = emb.shape[1]
    H_ = W_hh0.shape[1]
    Ls = L_ // _NSEG

    xt = x.T
    cs = jnp.concatenate([jnp.full((H_,), 0.5), jnp.full((H_,), 0.5),
                          jnp.ones((H_,)), jnp.full((H_,), 0.5)])
    w0 = (jnp.concatenate([W_ih0.T, 0.5 * W_hh0.T], axis=0)
          * cs).astype(jnp.bfloat16)
    w1 = (0.5 * jnp.concatenate([W_ih1.T, W_hh1.T], axis=0)
          * cs).astype(jnp.bfloat16)
    b0 = ((b_ih0 + b_hh0) * cs).reshape(1, -1)
    b1 = ((b_ih1 + b_hh1) * cs).reshape(1, -1)
    lwt = 0.5 * lin_w.T
    lb = lin_b.reshape(1, -1)

    segs = []
    for s in range(_NSEG):
        idx3d = xt[s * Ls:(s + 1) * Ls].reshape(_NW, -1, _CHUNK)
        segs.append(_sc_gather(emb, idx3d).reshape(Ls, B_, E_))

    bi, bf, bg, bo = jnp.split(b1.reshape(-1), 4)
    c1row = -(1.0 + jnp.tanh(bi)) * jnp.tanh(bg) / (1.0 + jnp.tanh(bf))
    zh = jnp.zeros((B_, H_), jnp.bfloat16)
    zc = jnp.zeros((B_, H_), jnp.float32)
    c1_0 = jnp.broadcast_to(c1row[None, :], (B_, H_)).astype(jnp.float32)
    state = (zh, zc, zh, c1_0)
    for s in range(_NSEG):
        *state, sig = _lstm_seg(segs[s], w0, w1, b0, b1, lwt, lb, state,
                                last=(s == _NSEG - 1))
    return sig[:, -1]


# speedup vs baseline: 1.3804x; 1.2109x over previous
import jax
import jax.numpy as jnp
from jax import lax
from jax.experimental import pallas as pl
from jax.experimental.pallas import tpu as pltpu
from jax.experimental.pallas import tpu_sc as plsc

_NC, _NS = 2, 16
_NW = _NC * _NS
_CHUNK = 128


def _sc_gather(emb, idx3d):
    nw, chunks_per_w, chunk = idx3d.shape
    n_chunks = nw * chunks_per_w
    E = emb.shape[1]

    def body(emb_hbm, idx_hbm, out_hbm, idx_v,
             buf0, buf1, g0, g1, o0, o1):
        wid = lax.axis_index("s") * _NC + lax.axis_index("c")
        base = wid * chunks_per_w
        pltpu.sync_copy(idx_hbm.at[wid], idx_v)
        bufs, gsems, osems = (buf0, buf1), (g0, g1), (o0, o1)

        def gather(c, j):
            return pltpu.make_async_copy(
                emb_hbm.at[idx_v.at[c]], bufs[j], gsems[j])

        def putout(c, j):
            return pltpu.make_async_copy(
                bufs[j], out_hbm.at[base + c], osems[j])

        gather(0, 0).start()
        gather(1, 1).start()

        def pair_step(p, carry):
            for j in range(2):
                c = 2 * p + j
                gather(c, j).wait()
                putout(c, j).start()
                putout(c, j).wait()

                @pl.when(c + 2 < chunks_per_w)
                def _():
                    gather(c + 2, j).start()
            return carry

        lax.fori_loop(0, chunks_per_w // 2, pair_step, 0)

    f = pl.kernel(
        body,
        out_type=jax.ShapeDtypeStruct((n_chunks, chunk, E), jnp.float32),
        mesh=plsc.VectorSubcoreMesh(core_axis_name="c", subcore_axis_name="s"),
        scratch_types=[
            pltpu.VMEM((chunks_per_w, chunk), jnp.int32),
            pltpu.VMEM((chunk, E), jnp.float32),
            pltpu.VMEM((chunk, E), jnp.float32),
            pltpu.SemaphoreType.DMA,
            pltpu.SemaphoreType.DMA,
            pltpu.SemaphoreType.DMA,
            pltpu.SemaphoreType.DMA,
        ],
    )
    return f(emb, idx3d)


def _sig(x):
    return 0.5 * jnp.tanh(0.5 * x) + 0.5


def _gates(gg, c, H_):
    i, f, g, o = jnp.split(gg, 4, axis=1)
    ti = jnp.tanh(i)
    tf = jnp.tanh(f)
    tg = jnp.tanh(g)
    to = jnp.tanh(o)
    c_new = 0.5 * ((1.0 + tf) * c + (1.0 + ti) * tg)
    h2 = (1.0 + to) * jnp.tanh(c_new)
    return c_new, h2


def _lstm_seg(embeds, w0, w1, b0, b1, state):
    L_, B_, E_ = embeds.shape
    H_ = w0.shape[1] // 4
    n_iter = L_ // 2

    def half(e, h0p, c0v, h1v, c1v, w0_ref, w1_ref, b0_ref, b1_ref):
        cat1 = jnp.concatenate([h0p, h1v], axis=1)
        g1 = jnp.dot(cat1, w1_ref[...],
                     preferred_element_type=jnp.float32) + b1_ref[...]
        c1n, h1n2 = _gates(g1, c1v, H_)

        cat0 = jnp.concatenate([e, h0p], axis=1)
        g0 = jnp.dot(cat0, w0_ref[...],
                     preferred_element_type=jnp.float32) + b0_ref[...]
        c0n, h0n2 = _gates(g0, c0v, H_)
        return h0n2.astype(jnp.bfloat16), c0n, h1n2.astype(jnp.bfloat16), c1n

    def body(e_ref, w0_ref, w1_ref, b0_ref, b1_ref,
             h0_in, c0_in, h1_in, c1_in,
             h0_out, c0_out, h1_out, c1_out,
             h0, c0, h1, c1):
        t = pl.program_id(0)

        @pl.when(t == 0)
        def _():
            h0[...] = h0_in[...]
            c0[...] = c0_in[...]
            h1[...] = h1_in[...]
            c1[...] = c1_in[...]

        e0 = e_ref[0].astype(jnp.bfloat16)
        e1 = e_ref[1].astype(jnp.bfloat16)
        h0a, c0a, h1a, c1a = half(e0, h0[...], c0[...], h1[...], c1[...],
                                  w0_ref, w1_ref, b0_ref, b1_ref)
        h0b, c0b, h1b, c1b = half(e1, h0a, c0a, h1a, c1a,
                                  w0_ref, w1_ref, b0_ref, b1_ref)
        h0[...] = h0b
        c0[...] = c0b
        h1[...] = h1b
        c1[...] = c1b

        @pl.when(t == n_iter - 1)
        def _():
            h0_out[...] = h0b
            c0_out[...] = c0b
            h1_out[...] = h1b
            c1_out[...] = c1b

    full = lambda shape: pl.BlockSpec(shape, lambda t: (0,) * len(shape))
    return pl.pallas_call(
        body,
        grid=(n_iter,),
        in_specs=[
            pl.BlockSpec((2, B_, E_), lambda t: (t, 0, 0)),
            full(w0.shape), full(w1.shape), full(b0.shape), full(b1.shape),
            full((B_, H_)), full((B_, H_)), full((B_, H_)), full((B_, H_)),
        ],
        out_specs=[full((B_, H_))] * 4,
        out_shape=[jax.ShapeDtypeStruct((B_, H_), jnp.bfloat16),
                   jax.ShapeDtypeStruct((B_, H_), jnp.float32),
                   jax.ShapeDtypeStruct((B_, H_), jnp.bfloat16),
                   jax.ShapeDtypeStruct((B_, H_), jnp.float32)],
        scratch_shapes=[pltpu.VMEM((B_, H_), jnp.bfloat16),
                        pltpu.VMEM((B_, H_), jnp.float32),
                        pltpu.VMEM((B_, H_), jnp.bfloat16),
                        pltpu.VMEM((B_, H_), jnp.float32)],
    )(embeds, w0, w1, b0, b1, *state)


def _lstm_drain(h0, h1, c1, w1, b1, lwt, lb):
    B_, H_ = c1.shape
    C_ = lwt.shape[1]

    def body(h0_ref, h1_ref, c1_ref, w1_ref, b1_ref, lw_ref, lb_ref, sig_ref):
        cat1 = jnp.concatenate([h0_ref[...], h1_ref[...]], axis=1)
        g1 = jnp.dot(cat1, w1_ref[...],
                     preferred_element_type=jnp.float32) + b1_ref[...]
        _, h1n2 = _gates(g1, c1_ref[...], H_)
        logits = jnp.dot(h1n2, lw_ref[...],
                         preferred_element_type=jnp.float32) + lb_ref[...]
        sig_ref[...] = _sig(logits)

    return pl.pallas_call(
        body,
        out_shape=jax.ShapeDtypeStruct((B_, C_), jnp.float32),
    )(h0, h1, c1, w1, b1, lwt, lb)


_NSEG = 5


def kernel(x, emb, W_ih0, W_hh0, b_ih0, b_hh0, W_ih1, W_hh1, b_ih1, b_hh1,
           lin_w, lin_b):
    B_, L_ = x.shape
    E_ = emb.shape[1]
    H_ = W_hh0.shape[1]
    Ls = L_ // _NSEG

    xt = x.T
    cs = jnp.concatenate([jnp.full((H_,), 0.5), jnp.full((H_,), 0.5),
                          jnp.ones((H_,)), jnp.full((H_,), 0.5)])
    w0 = (jnp.concatenate([W_ih0.T, 0.5 * W_hh0.T], axis=0)
          * cs).astype(jnp.bfloat16)
    w1 = (0.5 * jnp.concatenate([W_ih1.T, W_hh1.T], axis=0)
          * cs).astype(jnp.bfloat16)
    b0 = ((b_ih0 + b_hh0) * cs).reshape(1, -1)
    b1 = ((b_ih1 + b_hh1) * cs).reshape(1, -1)
    lwt = 0.5 * lin_w.T
    lb = lin_b.reshape(1, -1)

    segs = []
    for s in range(_NSEG):
        idx3d = xt[s * Ls:(s + 1) * Ls].reshape(_NW, -1, _CHUNK)
        segs.append(_sc_gather(emb, idx3d).reshape(Ls, B_, E_))

    bi, bf, bg, bo = jnp.split(b1.reshape(-1), 4)
    c1row = -(1.0 + jnp.tanh(bi)) * jnp.tanh(bg) / (1.0 + jnp.tanh(bf))
    zh = jnp.zeros((B_, H_), jnp.bfloat16)
    zc = jnp.zeros((B_, H_), jnp.float32)
    c1_0 = jnp.broadcast_to(c1row[None, :], (B_, H_)).astype(jnp.float32)
    state = (zh, zc, zh, c1_0)
    for s in range(_NSEG):
        state = _lstm_seg(segs[s], w0, w1, b0, b1, state)
    sig = _lstm_drain(state[0], state[2], state[3], w1, b1, lwt, lb)
    return sig[:, -1]


# speedup vs baseline: 1.4348x; 1.0394x over previous
import jax
import jax.numpy as jnp
from jax import lax
from jax.experimental import pallas as pl
from jax.experimental.pallas import tpu as pltpu
from jax.experimental.pallas import tpu_sc as plsc

_NC, _NS = 2, 16
_NW = _NC * _NS
_CHUNK = 128
_UNROLL = 4


def _sc_gather(emb, idx3d):
    nw, chunks_per_w, chunk = idx3d.shape
    n_chunks = nw * chunks_per_w
    E = emb.shape[1]

    def body(emb_hbm, idx_hbm, out_hbm, idx_v,
             buf0, buf1, g0, g1, o0, o1):
        wid = lax.axis_index("s") * _NC + lax.axis_index("c")
        base = wid * chunks_per_w
        pltpu.sync_copy(idx_hbm.at[wid], idx_v)
        bufs, gsems, osems = (buf0, buf1), (g0, g1), (o0, o1)

        def gather(c, j):
            return pltpu.make_async_copy(
                emb_hbm.at[idx_v.at[c]], bufs[j], gsems[j])

        def putout(c, j):
            return pltpu.make_async_copy(
                bufs[j], out_hbm.at[base + c], osems[j])

        gather(0, 0).start()
        gather(1, 1).start()

        def pair_step(p, carry):
            for j in range(2):
                c = 2 * p + j
                gather(c, j).wait()
                putout(c, j).start()
                putout(c, j).wait()

                @pl.when(c + 2 < chunks_per_w)
                def _():
                    gather(c + 2, j).start()
            return carry

        lax.fori_loop(0, chunks_per_w // 2, pair_step, 0)

    f = pl.kernel(
        body,
        out_type=jax.ShapeDtypeStruct((n_chunks, chunk, E), jnp.float32),
        mesh=plsc.VectorSubcoreMesh(core_axis_name="c", subcore_axis_name="s"),
        scratch_types=[
            pltpu.VMEM((chunks_per_w, chunk), jnp.int32),
            pltpu.VMEM((chunk, E), jnp.float32),
            pltpu.VMEM((chunk, E), jnp.float32),
            pltpu.SemaphoreType.DMA,
            pltpu.SemaphoreType.DMA,
            pltpu.SemaphoreType.DMA,
            pltpu.SemaphoreType.DMA,
        ],
    )
    return f(emb, idx3d)


def _sig(x):
    return 0.5 * jnp.tanh(0.5 * x) + 0.5


def _gates(gg, c, H_):
    i, f, g, o = jnp.split(gg, 4, axis=1)
    ti = jnp.tanh(i)
    tf = jnp.tanh(f)
    tg = jnp.tanh(g)
    to = jnp.tanh(o)
    c_new = 0.5 * ((1.0 + tf) * c + (1.0 + ti) * tg)
    h2 = (1.0 + to) * jnp.tanh(c_new)
    return c_new, h2


def _lstm_seg(embeds, w0, w1, b0, b1, state):
    L_, B_, E_ = embeds.shape
    H_ = w0.shape[1] // 4
    U = _UNROLL
    n_iter = L_ // U

    def half(e, h0p, c0v, h1v, c1v, w0_ref, w1_ref, b0_ref, b1_ref):
        cat1 = jnp.concatenate([h0p, h1v], axis=1)
        g1 = jnp.dot(cat1, w1_ref[...],
                     preferred_element_type=jnp.float32) + b1_ref[...]
        c1n, h1n2 = _gates(g1, c1v, H_)

        cat0 = jnp.concatenate([e, h0p], axis=1)
        g0 = jnp.dot(cat0, w0_ref[...],
                     preferred_element_type=jnp.float32) + b0_ref[...]
        c0n, h0n2 = _gates(g0, c0v, H_)
        return h0n2.astype(jnp.bfloat16), c0n, h1n2.astype(jnp.bfloat16), c1n

    def body(e_ref, w0_ref, w1_ref, b0_ref, b1_ref,
             h0_in, c0_in, h1_in, c1_in,
             h0_out, c0_out, h1_out, c1_out,
             h0, c0, h1, c1):
        t = pl.program_id(0)

        @pl.when(t == 0)
        def _():
            h0[...] = h0_in[...]
            c0[...] = c0_in[...]
            h1[...] = h1_in[...]
            c1[...] = c1_in[...]

        h0v, c0v, h1v, c1v = h0[...], c0[...], h1[...], c1[...]
        for u in range(U):
            e_u = e_ref[u].astype(jnp.bfloat16)
            h0v, c0v, h1v, c1v = half(e_u, h0v, c0v, h1v, c1v,
                                      w0_ref, w1_ref, b0_ref, b1_ref)
        h0[...] = h0v
        c0[...] = c0v
        h1[...] = h1v
        c1[...] = c1v

        @pl.when(t == n_iter - 1)
        def _():
            h0_out[...] = h0v
            c0_out[...] = c0v
            h1_out[...] = h1v
            c1_out[...] = c1v

    full = lambda shape: pl.BlockSpec(shape, lambda t: (0,) * len(shape))
    return pl.pallas_call(
        body,
        grid=(n_iter,),
        in_specs=[
            pl.BlockSpec((U, B_, E_), lambda t: (t, 0, 0)),
            full(w0.shape), full(w1.shape), full(b0.shape), full(b1.shape),
            full((B_, H_)), full((B_, H_)), full((B_, H_)), full((B_, H_)),
        ],
        out_specs=[full((B_, H_))] * 4,
        out_shape=[jax.ShapeDtypeStruct((B_, H_), jnp.bfloat16),
                   jax.ShapeDtypeStruct((B_, H_), jnp.float32),
                   jax.ShapeDtypeStruct((B_, H_), jnp.bfloat16),
                   jax.ShapeDtypeStruct((B_, H_), jnp.float32)],
        scratch_shapes=[pltpu.VMEM((B_, H_), jnp.bfloat16),
                        pltpu.VMEM((B_, H_), jnp.float32),
                        pltpu.VMEM((B_, H_), jnp.bfloat16),
                        pltpu.VMEM((B_, H_), jnp.float32)],
    )(embeds, w0, w1, b0, b1, *state)


def _lstm_drain(h0, h1, c1, w1, b1, lwt, lb):
    B_, H_ = c1.shape
    C_ = lwt.shape[1]

    def body(h0_ref, h1_ref, c1_ref, w1_ref, b1_ref, lw_ref, lb_ref, sig_ref):
        cat1 = jnp.concatenate([h0_ref[...], h1_ref[...]], axis=1)
        g1 = jnp.dot(cat1, w1_ref[...],
                     preferred_element_type=jnp.float32) + b1_ref[...]
        _, h1n2 = _gates(g1, c1_ref[...], H_)
        logits = jnp.dot(h1n2, lw_ref[...],
                         preferred_element_type=jnp.float32) + lb_ref[...]
        sig_ref[...] = _sig(logits)

    return pl.pallas_call(
        body,
        out_shape=jax.ShapeDtypeStruct((B_, C_), jnp.float32),
    )(h0, h1, c1, w1, b1, lwt, lb)


_NSEG = 5


def kernel(x, emb, W_ih0, W_hh0, b_ih0, b_hh0, W_ih1, W_hh1, b_ih1, b_hh1,
           lin_w, lin_b):
    B_, L_ = x.shape
    E_ = emb.shape[1]
    H_ = W_hh0.shape[1]
    Ls = L_ // _NSEG

    xt = x.T
    cs = jnp.concatenate([jnp.full((H_,), 0.5), jnp.full((H_,), 0.5),
                          jnp.ones((H_,)), jnp.full((H_,), 0.5)])
    w0 = (jnp.concatenate([W_ih0.T, 0.5 * W_hh0.T], axis=0)
          * cs).astype(jnp.bfloat16)
    w1 = (0.5 * jnp.concatenate([W_ih1.T, W_hh1.T], axis=0)
          * cs).astype(jnp.bfloat16)
    b0 = ((b_ih0 + b_hh0) * cs).reshape(1, -1)
    b1 = ((b_ih1 + b_hh1) * cs).reshape(1, -1)
    lwt = 0.5 * lin_w.T
    lb = lin_b.reshape(1, -1)

    segs = []
    for s in range(_NSEG):
        idx3d = xt[s * Ls:(s + 1) * Ls].reshape(_NW, -1, _CHUNK)
        segs.append(_sc_gather(emb, idx3d).reshape(Ls, B_, E_))

    bi, bf, bg, bo = jnp.split(b1.reshape(-1), 4)
    c1row = -(1.0 + jnp.tanh(bi)) * jnp.tanh(bg) / (1.0 + jnp.tanh(bf))
    zh = jnp.zeros((B_, H_), jnp.bfloat16)
    zc = jnp.zeros((B_, H_), jnp.float32)
    c1_0 = jnp.broadcast_to(c1row[None, :], (B_, H_)).astype(jnp.float32)
    state = (zh, zc, zh, c1_0)
    for s in range(_NSEG):
        state = _lstm_seg(segs[s], w0, w1, b0, b1, state)
    sig = _lstm_drain(state[0], state[2], state[3], w1, b1, lwt, lb)
    return sig[:, -1]


# speedup vs baseline: 1.4444x; 1.0067x over previous
import jax
import jax.numpy as jnp
from jax import lax
from jax.experimental import pallas as pl
from jax.experimental.pallas import tpu as pltpu
from jax.experimental.pallas import tpu_sc as plsc

_NC, _NS = 2, 16
_NW = _NC * _NS
_CHUNK = 128
_UNROLL = 8


def _sc_gather(emb, idx3d):
    nw, chunks_per_w, chunk = idx3d.shape
    n_chunks = nw * chunks_per_w
    E = emb.shape[1]

    def body(emb_hbm, idx_hbm, out_hbm, idx_v,
             buf0, buf1, g0, g1, o0, o1):
        wid = lax.axis_index("s") * _NC + lax.axis_index("c")
        base = wid * chunks_per_w
        pltpu.sync_copy(idx_hbm.at[wid], idx_v)
        bufs, gsems, osems = (buf0, buf1), (g0, g1), (o0, o1)

        def gather(c, j):
            return pltpu.make_async_copy(
                emb_hbm.at[idx_v.at[c]], bufs[j], gsems[j])

        def putout(c, j):
            return pltpu.make_async_copy(
                bufs[j], out_hbm.at[base + c], osems[j])

        gather(0, 0).start()
        gather(1, 1).start()

        def pair_step(p, carry):
            for j in range(2):
                c = 2 * p + j
                gather(c, j).wait()
                putout(c, j).start()
                putout(c, j).wait()

                @pl.when(c + 2 < chunks_per_w)
                def _():
                    gather(c + 2, j).start()
            return carry

        lax.fori_loop(0, chunks_per_w // 2, pair_step, 0)

    f = pl.kernel(
        body,
        out_type=jax.ShapeDtypeStruct((n_chunks, chunk, E), jnp.float32),
        mesh=plsc.VectorSubcoreMesh(core_axis_name="c", subcore_axis_name="s"),
        scratch_types=[
            pltpu.VMEM((chunks_per_w, chunk), jnp.int32),
            pltpu.VMEM((chunk, E), jnp.float32),
            pltpu.VMEM((chunk, E), jnp.float32),
            pltpu.SemaphoreType.DMA,
            pltpu.SemaphoreType.DMA,
            pltpu.SemaphoreType.DMA,
            pltpu.SemaphoreType.DMA,
        ],
    )
    return f(emb, idx3d)


def _sig(x):
    return 0.5 * jnp.tanh(0.5 * x) + 0.5


def _gates(gg, c, H_):
    i, f, g, o = jnp.split(gg, 4, axis=1)
    ti = jnp.tanh(i)
    tf = jnp.tanh(f)
    tg = jnp.tanh(g)
    to = jnp.tanh(o)
    c_new = 0.5 * ((1.0 + tf) * c + (1.0 + ti) * tg)
    h2 = (1.0 + to) * jnp.tanh(c_new)
    return c_new, h2


def _lstm_seg(embeds, w0, w1, b0, b1, state):
    L_, B_, E_ = embeds.shape
    H_ = w0.shape[1] // 4
    U = _UNROLL
    n_iter = L_ // U

    def half(e, h0p, c0v, h1v, c1v, w0_ref, w1_ref, b0_ref, b1_ref):
        cat1 = jnp.concatenate([h0p, h1v], axis=1)
        g1 = jnp.dot(cat1, w1_ref[...],
                     preferred_element_type=jnp.float32) + b1_ref[...]
        c1n, h1n2 = _gates(g1, c1v, H_)

        cat0 = jnp.concatenate([e, h0p], axis=1)
        g0 = jnp.dot(cat0, w0_ref[...],
                     preferred_element_type=jnp.float32) + b0_ref[...]
        c0n, h0n2 = _gates(g0, c0v, H_)
        return h0n2.astype(jnp.bfloat16), c0n, h1n2.astype(jnp.bfloat16), c1n

    def body(e_ref, w0_ref, w1_ref, b0_ref, b1_ref,
             h0_in, c0_in, h1_in, c1_in,
             h0_out, c0_out, h1_out, c1_out,
             h0, c0, h1, c1):
        t = pl.program_id(0)

        @pl.when(t == 0)
        def _():
            h0[...] = h0_in[...]
            c0[...] = c0_in[...]
            h1[...] = h1_in[...]
            c1[...] = c1_in[...]

        h0v, c0v, h1v, c1v = h0[...], c0[...], h1[...], c1[...]
        for u in range(U):
            e_u = e_ref[u].astype(jnp.bfloat16)
            h0v, c0v, h1v, c1v = half(e_u, h0v, c0v, h1v, c1v,
                                      w0_ref, w1_ref, b0_ref, b1_ref)
        h0[...] = h0v
        c0[...] = c0v
        h1[...] = h1v
        c1[...] = c1v

        @pl.when(t == n_iter - 1)
        def _():
            h0_out[...] = h0v
            c0_out[...] = c0v
            h1_out[...] = h1v
            c1_out[...] = c1v

    full = lambda shape: pl.BlockSpec(shape, lambda t: (0,) * len(shape))
    return pl.pallas_call(
        body,
        grid=(n_iter,),
        in_specs=[
            pl.BlockSpec((U, B_, E_), lambda t: (t, 0, 0)),
            full(w0.shape), full(w1.shape), full(b0.shape), full(b1.shape),
            full((B_, H_)), full((B_, H_)), full((B_, H_)), full((B_, H_)),
        ],
        out_specs=[full((B_, H_))] * 4,
        out_shape=[jax.ShapeDtypeStruct((B_, H_), jnp.bfloat16),
                   jax.ShapeDtypeStruct((B_, H_), jnp.float32),
                   jax.ShapeDtypeStruct((B_, H_), jnp.bfloat16),
                   jax.ShapeDtypeStruct((B_, H_), jnp.float32)],
        scratch_shapes=[pltpu.VMEM((B_, H_), jnp.bfloat16),
                        pltpu.VMEM((B_, H_), jnp.float32),
                        pltpu.VMEM((B_, H_), jnp.bfloat16),
                        pltpu.VMEM((B_, H_), jnp.float32)],
    )(embeds, w0, w1, b0, b1, *state)


def _lstm_drain(h0, h1, c1, w1, b1, lwt, lb):
    B_, H_ = c1.shape
    C_ = lwt.shape[1]

    def body(h0_ref, h1_ref, c1_ref, w1_ref, b1_ref, lw_ref, lb_ref, sig_ref):
        cat1 = jnp.concatenate([h0_ref[...], h1_ref[...]], axis=1)
        g1 = jnp.dot(cat1, w1_ref[...],
                     preferred_element_type=jnp.float32) + b1_ref[...]
        _, h1n2 = _gates(g1, c1_ref[...], H_)
        logits = jnp.dot(h1n2, lw_ref[...],
                         preferred_element_type=jnp.float32) + lb_ref[...]
        sig_ref[...] = _sig(logits)

    return pl.pallas_call(
        body,
        out_shape=jax.ShapeDtypeStruct((B_, C_), jnp.float32),
    )(h0, h1, c1, w1, b1, lwt, lb)


_NSEG = 5


def kernel(x, emb, W_ih0, W_hh0, b_ih0, b_hh0, W_ih1, W_hh1, b_ih1, b_hh1,
           lin_w, lin_b):
    B_, L_ = x.shape
    E_ = emb.shape[1]
    H_ = W_hh0.shape[1]
    Ls = L_ // _NSEG

    xt = x.T
    cs = jnp.concatenate([jnp.full((H_,), 0.5), jnp.full((H_,), 0.5),
                          jnp.ones((H_,)), jnp.full((H_,), 0.5)])
    w0 = (jnp.concatenate([W_ih0.T, 0.5 * W_hh0.T], axis=0)
          * cs).astype(jnp.bfloat16)
    w1 = (0.5 * jnp.concatenate([W_ih1.T, W_hh1.T], axis=0)
          * cs).astype(jnp.bfloat16)
    b0 = ((b_ih0 + b_hh0) * cs).reshape(1, -1)
    b1 = ((b_ih1 + b_hh1) * cs).reshape(1, -1)
    lwt = 0.5 * lin_w.T
    lb = lin_b.reshape(1, -1)

    segs = []
    for s in range(_NSEG):
        idx3d = xt[s * Ls:(s + 1) * Ls].reshape(_NW, -1, _CHUNK)
        segs.append(_sc_gather(emb, idx3d).reshape(Ls, B_, E_))

    bi, bf, bg, bo = jnp.split(b1.reshape(-1), 4)
    c1row = -(1.0 + jnp.tanh(bi)) * jnp.tanh(bg) / (1.0 + jnp.tanh(bf))
    zh = jnp.zeros((B_, H_), jnp.bfloat16)
    zc = jnp.zeros((B_, H_), jnp.float32)
    c1_0 = jnp.broadcast_to(c1row[None, :], (B_, H_)).astype(jnp.float32)
    state = (zh, zc, zh, c1_0)
    for s in range(_NSEG):
        state = _lstm_seg(segs[s], w0, w1, b0, b1, state)
    sig = _lstm_drain(state[0], state[2], state[3], w1, b1, lwt, lb)
    return sig[:, -1]
